# baseline (speedup 1.0000x reference)
import jax
import jax.numpy as jnp
from jax import lax
from jax.experimental import pallas as pl
from jax.experimental.pallas import tpu as pltpu
from jax.experimental.pallas import tpu_sc as plsc

B = 16384
S = 10
F = 32
NW = 32
B_PER_W = B // NW
PAIRS_PER_W = B_PER_W * S
IDXW = 128
N_STAGES = PAIRS_PER_W // IDXW
UID_ROWS = B_PER_W // IDXW
GROUPS = IDXW // 16

DOC_N = 1000000
USR_N = 100000


TBLK = 8192
TQ = TBLK // 4


def _detile_kernel(xT_ref, y_ref):
  eye = jnp.eye(F, dtype=jnp.float32)
  dims = (((0,), (0,)), ((), ()))
  ch = 256
  for k in range(TBLK // ch):
    x = xT_ref[:, pl.ds(k * ch, ch)]
    xt = jax.lax.dot_general(x, eye, dims)
    q, r0 = k // (TQ // ch), (k % (TQ // ch)) * ch
    y_ref[pl.ds(r0, ch), pl.ds(q * F, F)] = xt


def _detile(xT, n_items):
  grid = -(-n_items // TBLK)
  return pl.pallas_call(
      _detile_kernel,
      grid=(grid,),
      in_specs=[pl.BlockSpec((F, TBLK), lambda i: (0, i))],
      out_specs=pl.BlockSpec((TQ, 128), lambda i: (i, 0)),
      out_shape=jax.ShapeDtypeStruct((grid * TQ, 128), jnp.float32),
  )(xT)


def _permute_ids(ids):
  return ((ids & ~(TBLK - 1)) | ((ids & (TQ - 1)) << 2)
          | ((ids >> (TBLK.bit_length() - 3)) & 3))


def _compute_stage(buf, st, brow_v, usr_v, out_v):
  lane = jnp.arange(16, dtype=jnp.int32)
  for g in range(GROUPS):
    prow = lane + (g * 16)
    urow = brow_v[st, pl.ds(g * 16, 16)]
    accs = [jnp.zeros((16,), jnp.float32) for _ in range(4)]
    for f in range(F):
      colv = (lane + f) & (F - 1)
      dv = plsc.load_gather(buf, [prow, colv])
      uv = plsc.load_gather(usr_v, [urow, colv])
      accs[f % 4] = accs[f % 4] + dv * uv
    acc = (accs[0] + accs[1]) + (accs[2] + accs[3])
    out_v[pl.ds(st * IDXW + g * 16, 16)] = acc


def _sc_kernel(itT_hbm, uid_hbm, doc_hbm, usr_hbm, out_hbm,
               it_v, uid_v, did_v, brow_v, usr_v, doc0, doc1, out_v,
               sem_u, sem_e, sem_o):
  wid = lax.axis_index("s") * 2 + lax.axis_index("c")
  wb = wid * B_PER_W

  pltpu.sync_copy(itT_hbm.at[:, pl.ds(wb, B_PER_W)], it_v)
  pltpu.sync_copy(uid_hbm.at[pl.ds(wb, B_PER_W)], uid_v)

  @pl.loop(0, B_PER_W // 16)
  def _uperm(i):
    uv = uid_v[pl.ds(i * 16, 16)]
    uid_v[pl.ds(i * 16, 16)] = _permute_ids(uv)

  udescs = []
  for j in range(UID_ROWS):
    udescs.append(pltpu.async_copy(
        usr_hbm.at[uid_v.at[pl.ds(j * IDXW, IDXW)]],
        usr_v.at[pl.ds(j * IDXW, IDXW)], sem_u))

  lane = jnp.arange(16, dtype=jnp.int32)

  @pl.loop(0, N_STAGES)
  def _build(st):
    for g in range(GROUPS):
      pv = lane + (st * IDXW + g * 16)
      bv = (pv * 6554) >> 16
      sv = pv - bv * 10
      ids = plsc.load_gather(it_v, [sv, bv])
      did_v[st, pl.ds(g * 16, 16)] = _permute_ids(ids)
      brow_v[st, pl.ds(g * 16, 16)] = bv

  pltpu.async_copy(doc_hbm.at[did_v.at[0]], doc0, sem_e)

  for d in udescs:
    d.wait()

  @pl.loop(0, N_STAGES // 2)
  def _body(i):
    s0 = i * 2
    d_odd = pltpu.async_copy(doc_hbm.at[did_v.at[s0 + 1]], doc1, sem_o)
    pltpu.make_async_copy(doc_hbm.at[did_v.at[s0]], doc0, sem_e).wait()
    _compute_stage(doc0, s0, brow_v, usr_v, out_v)

    @pl.when(i < N_STAGES // 2 - 1)
    def _fire_even():
      pltpu.async_copy(doc_hbm.at[did_v.at[s0 + 2]], doc0, sem_e)

    d_odd.wait()
    _compute_stage(doc1, s0 + 1, brow_v, usr_v, out_v)

  pltpu.sync_copy(out_v, out_hbm.at[pl.ds(wid * PAIRS_PER_W, PAIRS_PER_W)])


@jax.jit
def _run(item_ids, user_ids, docEmbed, userEmbed):
  mesh = plsc.VectorSubcoreMesh(core_axis_name="c", subcore_axis_name="s")
  params = pltpu.CompilerParams(
      needs_layout_passes=False, use_tc_tiling_on_sc=False)

  doc_rm = _detile(docEmbed.T, DOC_N).reshape(-1, F)
  usr_rm = _detile(userEmbed.T, USR_N).reshape(-1, F)

  flat = pl.kernel(
      _sc_kernel,
      out_type=jax.ShapeDtypeStruct((B * S,), jnp.float32),
      mesh=mesh,
      compiler_params=params,
      scratch_types=[
          pltpu.VMEM((S, B_PER_W), jnp.int32),
          pltpu.VMEM((B_PER_W,), jnp.int32),
          pltpu.VMEM((N_STAGES, IDXW), jnp.int32),
          pltpu.VMEM((N_STAGES, IDXW), jnp.int32),
          pltpu.VMEM((B_PER_W, F), jnp.float32),
          pltpu.VMEM((IDXW, F), jnp.float32),
          pltpu.VMEM((IDXW, F), jnp.float32),
          pltpu.VMEM((PAIRS_PER_W,), jnp.float32),
          pltpu.SemaphoreType.DMA,
          pltpu.SemaphoreType.DMA,
          pltpu.SemaphoreType.DMA,
      ],
  )(item_ids.astype(jnp.int32).T, user_ids.astype(jnp.int32),
    doc_rm, usr_rm)
  return flat.reshape(B, S)


def kernel(item_ids, user_ids, docEmbed, userEmbed):
  return _run(item_ids, user_ids, docEmbed, userEmbed)


# speedup vs baseline: 1.0469x; 1.0469x over previous
import jax
import jax.numpy as jnp
from jax import lax
from jax.experimental import pallas as pl
from jax.experimental.pallas import tpu as pltpu
from jax.experimental.pallas import tpu_sc as plsc

B = 16384
S = 10
F = 32
NW = 32
B_PER_W = B // NW
PAIRS_PER_W = B_PER_W * S
IDXW = 128
N_STAGES = PAIRS_PER_W // IDXW
UID_ROWS = B_PER_W // IDXW
GROUPS = IDXW // 16

DOC_N = 1000000
USR_N = 100000


TBLK = 16384
TQ = TBLK // 4


def _detile_kernel(xT_ref, y_ref):
  eye = jnp.eye(F, dtype=jnp.float32)
  dims = (((0,), (0,)), ((), ()))
  ch = 256
  for k in range(TBLK // ch):
    x = xT_ref[:, pl.ds(k * ch, ch)]
    hi = x.astype(jnp.bfloat16).astype(jnp.float32)
    mid = (x - hi).astype(jnp.bfloat16).astype(jnp.float32)
    xt = (jax.lax.dot_general(hi, eye, dims)
          + jax.lax.dot_general(mid, eye, dims))
    q, r0 = k // (TQ // ch), (k % (TQ // ch)) * ch
    y_ref[pl.ds(r0, ch), pl.ds(q * F, F)] = xt


def _detile(xT, n_items):
  grid = -(-n_items // TBLK)
  return pl.pallas_call(
      _detile_kernel,
      grid=(grid,),
      in_specs=[pl.BlockSpec((F, TBLK), lambda i: (0, i))],
      out_specs=pl.BlockSpec((TQ, 128), lambda i: (i, 0)),
      out_shape=jax.ShapeDtypeStruct((grid * TQ, 128), jnp.float32),
  )(xT)


def _permute_ids(ids):
  return ((ids & ~(TBLK - 1)) | ((ids & (TQ - 1)) << 2)
          | ((ids >> (TBLK.bit_length() - 3)) & 3))


def _compute_stage(buf, st, brow_v, usr_v, out_v):
  lane = jnp.arange(16, dtype=jnp.int32)
  for g in range(GROUPS):
    prow = lane + (g * 16)
    urow = brow_v[st, pl.ds(g * 16, 16)]
    accs = [jnp.zeros((16,), jnp.float32) for _ in range(4)]
    for f in range(F):
      colv = (lane + f) & (F - 1)
      dv = plsc.load_gather(buf, [prow, colv])
      uv = plsc.load_gather(usr_v, [urow, colv])
      accs[f % 4] = accs[f % 4] + dv * uv
    acc = (accs[0] + accs[1]) + (accs[2] + accs[3])
    out_v[pl.ds(st * IDXW + g * 16, 16)] = acc


def _sc_kernel(itT_hbm, uid_hbm, doc_hbm, usr_hbm, out_hbm,
               it_v, uid_v, did_v, brow_v, usr_v, doc0, doc1, out_v,
               sem_u, sem_e, sem_o):
  wid = lax.axis_index("s") * 2 + lax.axis_index("c")
  wb = wid * B_PER_W

  pltpu.sync_copy(itT_hbm.at[:, pl.ds(wb, B_PER_W)], it_v)
  pltpu.sync_copy(uid_hbm.at[pl.ds(wb, B_PER_W)], uid_v)

  @pl.loop(0, B_PER_W // 16)
  def _uperm(i):
    uv = uid_v[pl.ds(i * 16, 16)]
    uid_v[pl.ds(i * 16, 16)] = _permute_ids(uv)

  udescs = []
  for j in range(UID_ROWS):
    udescs.append(pltpu.async_copy(
        usr_hbm.at[uid_v.at[pl.ds(j * IDXW, IDXW)]],
        usr_v.at[pl.ds(j * IDXW, IDXW)], sem_u))

  lane = jnp.arange(16, dtype=jnp.int32)

  @pl.loop(0, N_STAGES)
  def _build(st):
    for g in range(GROUPS):
      pv = lane + (st * IDXW + g * 16)
      bv = (pv * 6554) >> 16
      sv = pv - bv * 10
      ids = plsc.load_gather(it_v, [sv, bv])
      did_v[st, pl.ds(g * 16, 16)] = _permute_ids(ids)
      brow_v[st, pl.ds(g * 16, 16)] = bv

  pltpu.async_copy(doc_hbm.at[did_v.at[0]], doc0, sem_e)

  for d in udescs:
    d.wait()

  @pl.loop(0, N_STAGES // 2)
  def _body(i):
    s0 = i * 2
    d_odd = pltpu.async_copy(doc_hbm.at[did_v.at[s0 + 1]], doc1, sem_o)
    pltpu.make_async_copy(doc_hbm.at[did_v.at[s0]], doc0, sem_e).wait()
    _compute_stage(doc0, s0, brow_v, usr_v, out_v)

    @pl.when(i < N_STAGES // 2 - 1)
    def _fire_even():
      pltpu.async_copy(doc_hbm.at[did_v.at[s0 + 2]], doc0, sem_e)

    d_odd.wait()
    _compute_stage(doc1, s0 + 1, brow_v, usr_v, out_v)

  pltpu.sync_copy(out_v, out_hbm.at[pl.ds(wid * PAIRS_PER_W, PAIRS_PER_W)])


@jax.jit
def _run(item_ids, user_ids, docEmbed, userEmbed):
  mesh = plsc.VectorSubcoreMesh(core_axis_name="c", subcore_axis_name="s")
  params = pltpu.CompilerParams(
      needs_layout_passes=False, use_tc_tiling_on_sc=False)

  doc_rm = _detile(docEmbed.T, DOC_N).reshape(-1, F)
  usr_rm = _detile(userEmbed.T, USR_N).reshape(-1, F)

  flat = pl.kernel(
      _sc_kernel,
      out_type=jax.ShapeDtypeStruct((B * S,), jnp.float32),
      mesh=mesh,
      compiler_params=params,
      scratch_types=[
          pltpu.VMEM((S, B_PER_W), jnp.int32),
          pltpu.VMEM((B_PER_W,), jnp.int32),
          pltpu.VMEM((N_STAGES, IDXW), jnp.int32),
          pltpu.VMEM((N_STAGES, IDXW), jnp.int32),
          pltpu.VMEM((B_PER_W, F), jnp.float32),
          pltpu.VMEM((IDXW, F), jnp.float32),
          pltpu.VMEM((IDXW, F), jnp.float32),
          pltpu.VMEM((PAIRS_PER_W,), jnp.float32),
          pltpu.SemaphoreType.DMA,
          pltpu.SemaphoreType.DMA,
          pltpu.SemaphoreType.DMA,
      ],
  )(item_ids.astype(jnp.int32).T, user_ids.astype(jnp.int32),
    doc_rm, usr_rm)
  return flat.reshape(B, S)


def kernel(item_ids, user_ids, docEmbed, userEmbed):
  return _run(item_ids, user_ids, docEmbed, userEmbed)
